# baseline (speedup 1.0000x reference)
import functools

import jax
import jax.numpy as jnp
from jax import lax
from jax.experimental import pallas as pl
from jax.experimental.pallas import tpu as pltpu
from jax.experimental.pallas import tpu_sc as plsc

_DIM = 768
_P = 8
_N = 1024
_H = 32
_B = 8
_ROWS = _B * _N
_NC = 2
_NS = 16
_NW = _NC * _NS
_RPW = _ROWS // _NW
_LANES = 16
_CH = _DIM // _LANES


def _prep_kernel(x_ref, rp_ref, Wv_ref, bv_ref, Wcat_ref, bcat_ref,
                 v_ref, idx_ref, w_ref):
    b = pl.program_id(0)
    x = x_ref[0]

    value = jnp.dot(x, Wv_ref[...], preferred_element_type=jnp.float32)
    v_ref[0] = value + bv_ref[...]

    cat = jnp.dot(x, Wcat_ref[...], preferred_element_type=jnp.float32)
    cat = cat + bcat_ref[...]
    offx = cat[:, 0:_P]
    offy = cat[:, _P:2 * _P]
    awl = cat[:, 2 * _P:3 * _P]

    m = jnp.max(awl, axis=-1, keepdims=True)
    e = jnp.exp(awl - m)
    aw = e / jnp.sum(e, axis=-1, keepdims=True)

    rp = rp_ref[0]
    scale = (_H - 1) * 0.5
    gx = (jnp.clip(rp[:, 0:1] + offx, -1.0, 1.0) + 1.0) * scale
    gy = (jnp.clip(rp[:, 1:2] + offy, -1.0, 1.0) + 1.0) * scale

    x0 = jnp.floor(gx)
    y0 = jnp.floor(gy)
    fx = gx - x0
    fy = gy - y0
    x0i = x0.astype(jnp.int32)
    y0i = y0.astype(jnp.int32)
    x1i = jnp.minimum(x0i + 1, _H - 1)
    y1i = jnp.minimum(y0i + 1, _H - 1)

    base = b * _N
    i00 = base + y0i * _H + x0i
    i01 = base + y0i * _H + x1i
    i10 = base + y1i * _H + x0i
    i11 = base + y1i * _H + x1i
    idx_ref[0] = jnp.concatenate([i00, i01, i10, i11], axis=1)

    w00 = aw * (1.0 - fx) * (1.0 - fy)
    w01 = aw * fx * (1.0 - fy)
    w10 = aw * (1.0 - fx) * fy
    w11 = aw * fx * fy
    w = jnp.concatenate([w00, w01, w10, w11], axis=1)
    pieces = [
        jnp.broadcast_to(w[:, g:g + 1], (_N, _LANES)) for g in range(4 * _P)
    ]
    w_ref[0] = jnp.concatenate(pieces, axis=1)


def _sc_sample(v_hbm, idx_hbm, w_hbm, out_hbm, idx_v, w_v, rows_v, acc_v, sem):
    wid = lax.axis_index("s") * _NC + lax.axis_index("c")
    base = wid * _RPW

    def body(i, carry):
        r = base + i
        pltpu.sync_copy(idx_hbm.at[r], idx_v)
        pltpu.sync_copy(w_hbm.at[r], w_v)
        pltpu.async_copy(v_hbm.at[idx_v], rows_v, sem).wait()
        wspl = [w_v[pl.ds(g * _LANES, _LANES)] for g in range(4 * _P)]
        for c in range(_CH):
            acc = wspl[0] * rows_v[0, pl.ds(c * _LANES, _LANES)]
            for g in range(1, 4 * _P):
                acc = acc + wspl[g] * rows_v[g, pl.ds(c * _LANES, _LANES)]
            acc_v[pl.ds(c * _LANES, _LANES)] = acc
        pltpu.sync_copy(acc_v, out_hbm.at[r])
        return carry

    lax.fori_loop(0, _RPW, body, 0)


def _out_kernel(s_ref, Wo_ref, bo_ref, out_ref):
    out = jnp.dot(s_ref[...], Wo_ref[...], preferred_element_type=jnp.float32)
    out_ref[0] = out + bo_ref[...]


def kernel(x, ref_points, Wv, bv, Woff, boff, Waw, baw, Wo, bo):
    B, N, C = x.shape
    Woff3 = Woff.reshape(C, _P, 2)
    Wcat = jnp.concatenate([Woff3[:, :, 0], Woff3[:, :, 1], Waw], axis=1)
    boff3 = boff.reshape(_P, 2)
    bcat = jnp.concatenate([boff3[:, 0], boff3[:, 1], baw]).reshape(1, 3 * _P)

    value, idx, w = pl.pallas_call(
        _prep_kernel,
        grid=(B,),
        in_specs=[
            pl.BlockSpec((1, N, C), lambda b: (b, 0, 0)),
            pl.BlockSpec((1, N, 2), lambda b: (b, 0, 0)),
            pl.BlockSpec((C, C), lambda b: (0, 0)),
            pl.BlockSpec((1, C), lambda b: (0, 0)),
            pl.BlockSpec((C, 3 * _P), lambda b: (0, 0)),
            pl.BlockSpec((1, 3 * _P), lambda b: (0, 0)),
        ],
        out_specs=[
            pl.BlockSpec((1, N, C), lambda b: (b, 0, 0)),
            pl.BlockSpec((1, N, 4 * _P), lambda b: (b, 0, 0)),
            pl.BlockSpec((1, N, 4 * _P * _LANES), lambda b: (b, 0, 0)),
        ],
        out_shape=[
            jax.ShapeDtypeStruct((B, N, C), jnp.float32),
            jax.ShapeDtypeStruct((B, N, 4 * _P), jnp.int32),
            jax.ShapeDtypeStruct((B, N, 4 * _P * _LANES), jnp.float32),
        ],
    )(x, ref_points, Wv, bv.reshape(1, C), Wcat, bcat)

    value2 = value.reshape(_ROWS, C)
    idx2 = idx.reshape(_ROWS, 4 * _P)
    w2 = w.reshape(_ROWS, 4 * _P * _LANES)

    mesh = plsc.VectorSubcoreMesh(core_axis_name="c", subcore_axis_name="s")
    sampled = pl.kernel(
        _sc_sample,
        out_type=jax.ShapeDtypeStruct((_ROWS, C), jnp.float32),
        mesh=mesh,
        scratch_types=[
            pltpu.VMEM((4 * _P,), jnp.int32),
            pltpu.VMEM((4 * _P * _LANES,), jnp.float32),
            pltpu.VMEM((4 * _P, C), jnp.float32),
            pltpu.VMEM((C,), jnp.float32),
            pltpu.SemaphoreType.DMA,
        ],
    )(value2, idx2, w2)

    out = pl.pallas_call(
        _out_kernel,
        grid=(B,),
        in_specs=[
            pl.BlockSpec((N, C), lambda b: (b, 0)),
            pl.BlockSpec((C, C), lambda b: (0, 0)),
            pl.BlockSpec((1, C), lambda b: (0, 0)),
        ],
        out_specs=pl.BlockSpec((1, N, C), lambda b: (b, 0, 0)),
        out_shape=jax.ShapeDtypeStruct((B, N, C), jnp.float32),
    )(sampled, Wo, bo.reshape(1, C))
    return out


# speedup vs baseline: 28.8989x; 28.8989x over previous
import jax
import jax.numpy as jnp
from jax.experimental import pallas as pl

_DIM = 768
_P = 8
_N = 1024
_H = 32


def _dsaam_kernel(x_ref, rp_ref, Wv_ref, bv_ref, Wcat_ref, bcat_ref, Wo_ref,
                  bo_ref, out_ref):
    x = x_ref[0]

    value = jnp.dot(x.astype(jnp.bfloat16), Wv_ref[...],
                    preferred_element_type=jnp.float32)
    value = value + bv_ref[...]

    cat = jnp.dot(x, Wcat_ref[...], preferred_element_type=jnp.float32)
    cat = cat + bcat_ref[...]
    catT = cat.T
    offxT = catT[0:_P]
    offyT = catT[_P:2 * _P]
    awlT = catT[2 * _P:3 * _P]

    m = jnp.max(awlT, axis=0, keepdims=True)
    e = jnp.exp(awlT - m)
    awT = e / jnp.sum(e, axis=0, keepdims=True)

    rpT = rp_ref[0].T
    scale = (_H - 1) * 0.5
    gxT = (jnp.clip(rpT[0:1] + offxT, -1.0, 1.0) + 1.0) * scale
    gyT = (jnp.clip(rpT[1:2] + offyT, -1.0, 1.0) + 1.0) * scale

    grow = jax.lax.broadcasted_iota(jnp.int32, (_H, 1), 0).astype(jnp.float32)

    St3 = jnp.zeros((_H, _H, _N), jnp.bfloat16)
    for p in range(_P):
        Xp = jnp.maximum(0.0, 1.0 - jnp.abs(gxT[p:p + 1] - grow))
        Yp = jnp.maximum(0.0, 1.0 - jnp.abs(gyT[p:p + 1] - grow))
        Yp = awT[p:p + 1] * Yp
        St3 = St3 + Yp.astype(jnp.bfloat16)[:, None, :] * \
            Xp.astype(jnp.bfloat16)[None, :, :]
    St = St3.reshape(_N, _N)

    sampled = jax.lax.dot_general(St, value.astype(jnp.bfloat16),
                                  (((0,), (0,)), ((), ())),
                                  preferred_element_type=jnp.float32)
    out = jnp.dot(sampled.astype(jnp.bfloat16), Wo_ref[...],
                  preferred_element_type=jnp.float32)
    out_ref[0] = out + bo_ref[...]


def kernel(x, ref_points, Wv, bv, Woff, boff, Waw, baw, Wo, bo):
    B, N, C = x.shape
    Woff3 = Woff.reshape(C, _P, 2)
    Wcat = jnp.concatenate([Woff3[:, :, 0], Woff3[:, :, 1], Waw], axis=1)
    boff3 = boff.reshape(_P, 2)
    bcat = jnp.concatenate([boff3[:, 0], boff3[:, 1], baw]).reshape(1, 3 * _P)

    grid = (B,)
    out = pl.pallas_call(
        _dsaam_kernel,
        grid=grid,
        in_specs=[
            pl.BlockSpec((1, N, C), lambda b: (b, 0, 0)),
            pl.BlockSpec((1, N, 2), lambda b: (b, 0, 0)),
            pl.BlockSpec((C, C), lambda b: (0, 0)),
            pl.BlockSpec((1, C), lambda b: (0, 0)),
            pl.BlockSpec((C, 3 * _P), lambda b: (0, 0)),
            pl.BlockSpec((1, 3 * _P), lambda b: (0, 0)),
            pl.BlockSpec((C, C), lambda b: (0, 0)),
            pl.BlockSpec((1, C), lambda b: (0, 0)),
        ],
        out_specs=pl.BlockSpec((1, N, C), lambda b: (b, 0, 0)),
        out_shape=jax.ShapeDtypeStruct((B, N, C), jnp.float32),
    )(x, ref_points, Wv.astype(jnp.bfloat16), bv.reshape(1, C), Wcat, bcat,
      Wo.astype(jnp.bfloat16), bo.reshape(1, C))
    return out


# speedup vs baseline: 29.8109x; 1.0316x over previous
import jax
import jax.numpy as jnp
from jax.experimental import pallas as pl

_DIM = 768
_P = 8
_N = 1024
_H = 32


def _dsaam_kernel(x_ref, rp_ref, Wv_ref, bv_ref, Wcat_ref, bcat_ref, Wo_ref,
                  bo_ref, out_ref):
    x = x_ref[0]

    value = jnp.dot(x, Wv_ref[...], preferred_element_type=jnp.float32)
    value = value + bv_ref[...]

    cat = jnp.dot(x, Wcat_ref[...], preferred_element_type=jnp.float32)
    cat = cat + bcat_ref[...]
    catT = cat.T
    offxT = catT[0:_P]
    offyT = catT[_P:2 * _P]
    awlT = catT[2 * _P:3 * _P]

    m = jnp.max(awlT, axis=0, keepdims=True)
    e = jnp.exp(awlT - m)
    awT = e / jnp.sum(e, axis=0, keepdims=True)

    rpT = rp_ref[0].T
    scale = (_H - 1) * 0.5
    gxT = (jnp.clip(rpT[0:1] + offxT, -1.0, 1.0) + 1.0) * scale
    gyT = (jnp.clip(rpT[1:2] + offyT, -1.0, 1.0) + 1.0) * scale

    grow = jax.lax.broadcasted_iota(jnp.int32, (_H, 1), 0).astype(jnp.float32)

    St3 = jnp.zeros((_H, _H, _N), jnp.bfloat16)
    for p in range(_P):
        Xp = jnp.maximum(0.0, 1.0 - jnp.abs(gxT[p:p + 1] - grow))
        Yp = jnp.maximum(0.0, 1.0 - jnp.abs(gyT[p:p + 1] - grow))
        Yp = awT[p:p + 1] * Yp
        St3 = St3 + Yp.astype(jnp.bfloat16)[:, None, :] * \
            Xp.astype(jnp.bfloat16)[None, :, :]
    St = St3.reshape(_N, _N)

    sampled = jax.lax.dot_general(St, value.astype(jnp.bfloat16),
                                  (((0,), (0,)), ((), ())),
                                  preferred_element_type=jnp.float32)
    out = jnp.dot(sampled, Wo_ref[...], preferred_element_type=jnp.float32)
    out_ref[0] = out + bo_ref[...]


def kernel(x, ref_points, Wv, bv, Woff, boff, Waw, baw, Wo, bo):
    B, N, C = x.shape
    Woff3 = Woff.reshape(C, _P, 2)
    Wcat = jnp.concatenate([Woff3[:, :, 0], Woff3[:, :, 1], Waw], axis=1)
    boff3 = boff.reshape(_P, 2)
    bcat = jnp.concatenate([boff3[:, 0], boff3[:, 1], baw]).reshape(1, 3 * _P)

    grid = (B,)
    out = pl.pallas_call(
        _dsaam_kernel,
        grid=grid,
        in_specs=[
            pl.BlockSpec((1, N, C), lambda b: (b, 0, 0)),
            pl.BlockSpec((1, N, 2), lambda b: (b, 0, 0)),
            pl.BlockSpec((C, C), lambda b: (0, 0)),
            pl.BlockSpec((1, C), lambda b: (0, 0)),
            pl.BlockSpec((C, 3 * _P), lambda b: (0, 0)),
            pl.BlockSpec((1, 3 * _P), lambda b: (0, 0)),
            pl.BlockSpec((C, C), lambda b: (0, 0)),
            pl.BlockSpec((1, C), lambda b: (0, 0)),
        ],
        out_specs=pl.BlockSpec((1, N, C), lambda b: (b, 0, 0)),
        out_shape=jax.ShapeDtypeStruct((B, N, C), jnp.float32),
    )(x, ref_points, Wv, bv.reshape(1, C), Wcat, bcat, Wo, bo.reshape(1, C))
    return out


# speedup vs baseline: 30.3700x; 1.0188x over previous
import jax
import jax.numpy as jnp
from jax.experimental import pallas as pl

_DIM = 768
_P = 8
_N = 1024
_H = 32


def _dsaam_kernel(x_ref, rp_ref, Wv_ref, bv_ref, Wcat_ref, bcat_ref, Wo_ref,
                  bo_ref, out_ref):
  for s in range(2):
      x = x_ref[s]

      value = jnp.dot(x, Wv_ref[...], preferred_element_type=jnp.float32)
      value = value + bv_ref[...]

      cat = jnp.dot(x, Wcat_ref[...], preferred_element_type=jnp.float32)
      cat = cat + bcat_ref[...]
      catT = cat.T
      offxT = catT[0:_P]
      offyT = catT[_P:2 * _P]
      awlT = catT[2 * _P:3 * _P]

      m = jnp.max(awlT, axis=0, keepdims=True)
      e = jnp.exp(awlT - m)
      awT = e / jnp.sum(e, axis=0, keepdims=True)

      rpT = rp_ref[s].T
      scale = (_H - 1) * 0.5
      gxT = (jnp.clip(rpT[0:1] + offxT, -1.0, 1.0) + 1.0) * scale
      gyT = (jnp.clip(rpT[1:2] + offyT, -1.0, 1.0) + 1.0) * scale

      grow = jax.lax.broadcasted_iota(jnp.int32, (_H, 1), 0).astype(jnp.float32)

      St3 = jnp.zeros((_H, _H, _N), jnp.bfloat16)
      for p in range(_P):
          Xp = jnp.maximum(0.0, 1.0 - jnp.abs(gxT[p:p + 1] - grow))
          Yp = jnp.maximum(0.0, 1.0 - jnp.abs(gyT[p:p + 1] - grow))
          Yp = awT[p:p + 1] * Yp
          St3 = St3 + Yp.astype(jnp.bfloat16)[:, None, :] * \
              Xp.astype(jnp.bfloat16)[None, :, :]
      St = St3.reshape(_N, _N)

      sampled = jax.lax.dot_general(St, value.astype(jnp.bfloat16),
                                    (((0,), (0,)), ((), ())),
                                    preferred_element_type=jnp.float32)
      out = jnp.dot(sampled, Wo_ref[...], preferred_element_type=jnp.float32)
      out_ref[s] = out + bo_ref[...]


def kernel(x, ref_points, Wv, bv, Woff, boff, Waw, baw, Wo, bo):
    B, N, C = x.shape
    Woff3 = Woff.reshape(C, _P, 2)
    Wcat = jnp.concatenate([Woff3[:, :, 0], Woff3[:, :, 1], Waw], axis=1)
    boff3 = boff.reshape(_P, 2)
    bcat = jnp.concatenate([boff3[:, 0], boff3[:, 1], baw]).reshape(1, 3 * _P)

    grid = (B // 2,)
    out = pl.pallas_call(
        _dsaam_kernel,
        grid=grid,
        in_specs=[
            pl.BlockSpec((2, N, C), lambda b: (b, 0, 0)),
            pl.BlockSpec((2, N, 2), lambda b: (b, 0, 0)),
            pl.BlockSpec((C, C), lambda b: (0, 0)),
            pl.BlockSpec((1, C), lambda b: (0, 0)),
            pl.BlockSpec((C, 3 * _P), lambda b: (0, 0)),
            pl.BlockSpec((1, 3 * _P), lambda b: (0, 0)),
            pl.BlockSpec((C, C), lambda b: (0, 0)),
            pl.BlockSpec((1, C), lambda b: (0, 0)),
        ],
        out_specs=pl.BlockSpec((2, N, C), lambda b: (b, 0, 0)),
        out_shape=jax.ShapeDtypeStruct((B, N, C), jnp.float32),
    )(x, ref_points, Wv, bv.reshape(1, C), Wcat, bcat, Wo, bo.reshape(1, C))
    return out
